# baseline (speedup 1.0000x reference)
import functools

import jax
import jax.numpy as jnp
from jax import lax
from jax.experimental import pallas as pl
from jax.experimental.pallas import tpu as pltpu
from jax.experimental.pallas import tpu_sc as plsc

QB = 64
FB = 2048
LN = 128


def _sdiv(n, d):
    ds = jnp.where(jnp.abs(d) < 1e-12, jnp.where(d < 0, -1e-12, 1e-12), d)
    return n / ds


def _tri_chunk(t, px, py, pz):
    ax, ay, az = t[0:1], t[1:2], t[2:3]
    bx, by, bz = t[3:4], t[4:5], t[5:6]
    cx, cy, cz = t[6:7], t[7:8], t[8:9]
    abx, aby, abz = t[9:10], t[10:11], t[11:12]
    acx, acy, acz = t[12:13], t[13:14], t[14:15]
    apx, apy, apz = px - ax, py - ay, pz - az
    d1 = abx * apx + aby * apy + abz * apz
    d2 = acx * apx + acy * apy + acz * apz
    bpx, bpy, bpz = px - bx, py - by, pz - bz
    d3 = abx * bpx + aby * bpy + abz * bpz
    d4 = acx * bpx + acy * bpy + acz * bpz
    cpx, cpy, cpz = px - cx, py - cy, pz - cz
    d5 = abx * cpx + aby * cpy + abz * cpz
    d6 = acx * cpx + acy * cpy + acz * cpz
    vc = d1 * d4 - d3 * d2
    vb = d5 * d2 - d1 * d6
    va = d3 * d6 - d5 * d4
    denom = va + vb + vc
    dsafe = jnp.where(jnp.abs(denom) < 1e-12,
                      jnp.where(denom < 0, -1e-12, 1e-12), denom)
    v_in = vb / dsafe
    w_in = vc / dsafe
    u_in = 1.0 - v_in - w_in
    v_ab = _sdiv(d1, d1 - d3)
    w_ac = _sdiv(d2, d2 - d6)
    e1 = d4 - d3
    e2 = d5 - d6
    w_bc = _sdiv(e1, e1 + e2)
    zeros = jnp.zeros_like(d1)
    ones = jnp.ones_like(d1)

    def pick(cond, new, old):
        return tuple(jnp.where(cond, n, o) for n, o in zip(new, old))

    bc = (u_in, v_in, w_in)
    bc = pick((va <= 0) & (e1 >= 0) & (e2 >= 0), (zeros, 1.0 - w_bc, w_bc), bc)
    bc = pick((vb <= 0) & (d2 >= 0) & (d6 <= 0), (1.0 - w_ac, zeros, w_ac), bc)
    bc = pick((vc <= 0) & (d1 >= 0) & (d3 <= 0), (1.0 - v_ab, v_ab, zeros), bc)
    bc = pick((d6 >= 0) & (d5 <= d6), (zeros, zeros, ones), bc)
    bc = pick((d3 >= 0) & (d4 <= d3), (zeros, ones, zeros), bc)
    bc = pick((d1 <= 0) & (d2 <= 0), (ones, zeros, zeros), bc)
    u, v, w = bc
    qx = ax * u + bx * v + cx * w
    qy = ay * u + by * v + cy * w
    qz = az * u + bz * v + cz * w
    dx = qx - px
    dy = qy - py
    dz = qz - pz
    d2sq = dx * dx + dy * dy + dz * dz
    return d2sq, u, v, w


def _search_body(tris_ref, pts_ref, idx_ref, u_ref, v_ref, w_ref,
                 bd_s, bf_s, bu_s, bv_s, bw_s, *, nfaces, nq):
    b = pl.program_id(0)
    fb = pl.program_id(1)
    qi = pl.program_id(2)
    nf = pl.num_programs(1)
    qsl = pl.ds(qi * QB, QB)

    pts = pts_ref[0]
    px = pts[:, 0:1]
    py = pts[:, 1:2]
    pz = pts[:, 2:3]

    first = fb == 0
    bd = jnp.where(first, jnp.float32(1e30), bd_s[qsl])
    bf = jnp.where(first, 0, bf_s[qsl])
    bu = jnp.where(first, 0.0, bu_s[qsl])
    bv = jnp.where(first, 0.0, bv_s[qsl])
    bw = jnp.where(first, 0.0, bw_s[qsl])
    for c in range(FB // LN):
        t = tris_ref[0, :, c * LN:(c + 1) * LN]
        d2sq, u, v, w = _tri_chunk(t, px, py, pz)
        upd = d2sq < bd
        bd = jnp.where(upd, d2sq, bd)
        bf = jnp.where(upd, fb * (FB // LN) + c, bf)
        bu = jnp.where(upd, u, bu)
        bv = jnp.where(upd, v, bv)
        bw = jnp.where(upd, w, bw)
    bd_s[qsl] = bd
    bf_s[qsl] = bf
    bu_s[qsl] = bu
    bv_s[qsl] = bv
    bw_s[qsl] = bw

    @pl.when(fb == nf - 1)
    def _():
        lane = lax.broadcasted_iota(jnp.int32, (QB, LN), 1)
        ff = bf * LN + lane
        m = jnp.min(bd, axis=1, keepdims=True)
        fbig = jnp.where(bd == m, ff, jnp.int32(2147483647))
        fsel = jnp.min(fbig, axis=1, keepdims=True)
        sel = ff == fsel
        ub = jnp.sum(jnp.where(sel, bu, 0.0), axis=1, keepdims=True)
        vb_ = jnp.sum(jnp.where(sel, bv, 0.0), axis=1, keepdims=True)
        wb_ = jnp.sum(jnp.where(sel, bw, 0.0), axis=1, keepdims=True)
        idx_ref[0] = fsel + b * nfaces
        u_ref[0] = jnp.clip(ub, 0.0, 1.0)
        v_ref[0] = jnp.clip(vb_, 0.0, 1.0)
        w_ref[0] = jnp.clip(wb_, 0.0, 1.0)


def _search(tris_t, points):
    bsz, _, nfaces = tris_t.shape
    nq = points.shape[1]
    grid = (bsz, nfaces // FB, nq // QB)
    return pl.pallas_call(
        functools.partial(_search_body, nfaces=nfaces, nq=nq),
        grid=grid,
        in_specs=[
            pl.BlockSpec((1, 15, FB), lambda b, f, q: (b, 0, f)),
            pl.BlockSpec((1, QB, 3), lambda b, f, q: (b, q, 0)),
        ],
        out_specs=[
            pl.BlockSpec((1, QB, 1), lambda b, f, q: (b, q, 0)),
            pl.BlockSpec((1, QB, 1), lambda b, f, q: (b, q, 0)),
            pl.BlockSpec((1, QB, 1), lambda b, f, q: (b, q, 0)),
            pl.BlockSpec((1, QB, 1), lambda b, f, q: (b, q, 0)),
        ],
        out_shape=[
            jax.ShapeDtypeStruct((bsz, nq, 1), jnp.int32),
            jax.ShapeDtypeStruct((bsz, nq, 1), jnp.float32),
            jax.ShapeDtypeStruct((bsz, nq, 1), jnp.float32),
            jax.ShapeDtypeStruct((bsz, nq, 1), jnp.float32),
        ],
        scratch_shapes=[
            pltpu.VMEM((nq, LN), jnp.float32),
            pltpu.VMEM((nq, LN), jnp.int32),
            pltpu.VMEM((nq, LN), jnp.float32),
            pltpu.VMEM((nq, LN), jnp.float32),
            pltpu.VMEM((nq, LN), jnp.float32),
        ],
    )(tris_t, points)


NW = 32
GL = 16


def _combine_body(tri9, nrm9, cm9, fc3, idxf, uf, vf, wf, pxf, pyf, pzf,
                  rx_o, ry_o, rz_o, nx_o, ny_o, nz_o, cx_o, cy_o, cz_o, fid_o,
                  idx_v, u_v, v_v, w_v, px_v, py_v, pz_v, idx9, idx3,
                  cols_v, fcols_v,
                  orx, ory, orz, onx, ony, onz, ocx, ocy, ocz, ofid, sem,
                  *, chunk):
    cid = lax.axis_index("c")
    sid = lax.axis_index("s")
    wid = sid * 2 + cid
    base = wid * chunk
    pltpu.sync_copy(idxf.at[pl.ds(base, chunk)], idx_v)
    pltpu.sync_copy(uf.at[pl.ds(base, chunk)], u_v)
    pltpu.sync_copy(vf.at[pl.ds(base, chunk)], v_v)
    pltpu.sync_copy(wf.at[pl.ds(base, chunk)], w_v)
    pltpu.sync_copy(pxf.at[pl.ds(base, chunk)], px_v)
    pltpu.sync_copy(pyf.at[pl.ds(base, chunk)], py_v)
    pltpu.sync_copy(pzf.at[pl.ds(base, chunk)], pz_v)
    ngr = chunk // GL
    for g in range(ngr):
        sl = pl.ds(g * GL, GL)
        iv9 = idx_v[sl] * 9
        for c in range(9):
            idx9[c, sl] = iv9 + c
        iv3 = idx_v[sl] * 3
        for c in range(3):
            idx3[c, sl] = iv3 + c
    copies = []
    for s, src in enumerate((tri9, nrm9, cm9)):
        for c in range(9):
            copies.append(pltpu.async_copy(
                src.at[idx9.at[c]], cols_v.at[s * 9 + c], sem))
    for c in range(3):
        copies.append(pltpu.async_copy(fc3.at[idx3.at[c]], fcols_v.at[c], sem))
    for cp in copies:
        cp.wait()
    for g in range(ngr):
        sl = pl.ds(g * GL, GL)
        u16 = u_v[sl]
        v16 = v_v[sl]
        w16 = w_v[sl]

        def gcol(col):
            return cols_v[col, sl]

        p16 = (px_v[sl], py_v[sl], pz_v[sl])
        routs = (orx, ory, orz)
        for k in range(3):
            av = gcol(k)
            bv = gcol(3 + k)
            cv = gcol(6 + k)
            routs[k][sl] = (av * u16 + bv * v16 + cv * w16) - p16[k]
        nouts = (onx, ony, onz)
        for k in range(3):
            av = gcol(9 + k)
            bv = gcol(12 + k)
            cv = gcol(15 + k)
            nouts[k][sl] = av * u16 + bv * v16 + cv * w16
        couts = (ocx, ocy, ocz)
        for k in range(3):
            av = gcol(18 + k)
            bv = gcol(21 + k)
            cv = gcol(24 + k)
            couts[k][sl] = av * u16 + bv * v16 + cv * w16
        f0 = fcols_v[0, sl]
        f1 = fcols_v[1, sl]
        f2 = fcols_v[2, sl]
        m01 = jnp.maximum(u16, v16)
        ofid[sl] = jnp.where(w16 > m01, f2, jnp.where(v16 > u16, f1, f0))
    pltpu.sync_copy(orx, rx_o.at[pl.ds(base, chunk)])
    pltpu.sync_copy(ory, ry_o.at[pl.ds(base, chunk)])
    pltpu.sync_copy(orz, rz_o.at[pl.ds(base, chunk)])
    pltpu.sync_copy(onx, nx_o.at[pl.ds(base, chunk)])
    pltpu.sync_copy(ony, ny_o.at[pl.ds(base, chunk)])
    pltpu.sync_copy(onz, nz_o.at[pl.ds(base, chunk)])
    pltpu.sync_copy(ocx, cx_o.at[pl.ds(base, chunk)])
    pltpu.sync_copy(ocy, cy_o.at[pl.ds(base, chunk)])
    pltpu.sync_copy(ocz, cz_o.at[pl.ds(base, chunk)])
    pltpu.sync_copy(ofid, fid_o.at[pl.ds(base, chunk)])


def _combine(tri9, nrm9, cm9, fc3, idxf, uf, vf, wf, pxf, pyf, pzf):
    bq = idxf.shape[0]
    chunk = bq // NW
    f32 = jnp.float32
    out_type = [jax.ShapeDtypeStruct((bq,), f32)] * 9 + [
        jax.ShapeDtypeStruct((bq,), jnp.int32)]
    scratch = (
        [pltpu.VMEM((chunk,), jnp.int32)]
        + [pltpu.VMEM((chunk,), f32)] * 6
        + [pltpu.VMEM((9, chunk), jnp.int32)]
        + [pltpu.VMEM((3, chunk), jnp.int32)]
        + [pltpu.VMEM((27, chunk), f32)]
        + [pltpu.VMEM((3, chunk), jnp.int32)]
        + [pltpu.VMEM((chunk,), f32)] * 9
        + [pltpu.VMEM((chunk,), jnp.int32)]
        + [pltpu.SemaphoreType.DMA]
    )
    run = pl.kernel(
        functools.partial(_combine_body, chunk=chunk),
        out_type=out_type,
        mesh=plsc.VectorSubcoreMesh(core_axis_name="c", subcore_axis_name="s"),
        scratch_types=scratch,
    )
    return run(tri9, nrm9, cm9, fc3, idxf, uf, vf, wf, pxf, pyf, pzf)


def kernel(triangles, points, normals, cmaps, faces):
    bsz, nfaces = triangles.shape[:2]
    nq = points.shape[1]
    bq = bsz * nq
    t9 = triangles.reshape(bsz, nfaces, 9)
    tris_t = jnp.concatenate(
        [t9, t9[:, :, 3:6] - t9[:, :, 0:3], t9[:, :, 6:9] - t9[:, :, 0:3]],
        axis=2).transpose(0, 2, 1)
    idx_o, u_o, v_o, w_o = _search(tris_t, points)
    nbf = bsz * nfaces
    idxf = idx_o.reshape(bq)
    uf = u_o.reshape(bq)
    vf = v_o.reshape(bq)
    wf = w_o.reshape(bq)
    pxf = points[..., 0].reshape(bq)
    pyf = points[..., 1].reshape(bq)
    pzf = points[..., 2].reshape(bq)
    outs = _combine(triangles.reshape(nbf * 9), normals.reshape(nbf * 9),
                    cmaps.reshape(nbf * 9), faces.reshape(nbf * 3),
                    idxf, uf, vf, wf, pxf, pyf, pzf)
    rx, ry, rz, nx, ny, nz, cxo, cyo, czo, fid = outs
    residual = jnp.stack([rx, ry, rz], axis=-1).reshape(bsz, nq, 3)
    closest_normals = jnp.stack([nx, ny, nz], axis=-1).reshape(bsz, nq, 3)
    closest_cmaps = jnp.stack([cxo, cyo, czo], axis=-1).reshape(bsz, nq, 3)
    closest_idx = fid.reshape(bsz, nq)
    return residual, closest_normals, closest_cmaps, closest_idx


# speedup vs baseline: 1.1806x; 1.1806x over previous
import functools

import jax
import jax.numpy as jnp
from jax import lax
from jax.experimental import pallas as pl
from jax.experimental.pallas import tpu as pltpu
from jax.experimental.pallas import tpu_sc as plsc

QB = 64
FB = 2048
LN = 128


def _sdiv(n, d):
    ds = jnp.where(jnp.abs(d) < 1e-12, jnp.where(d < 0, -1e-12, 1e-12), d)
    return n / ds


def _tri_chunk(t, px, py, pz):
    ax, ay, az = t[0:1], t[1:2], t[2:3]
    bx, by, bz = t[3:4], t[4:5], t[5:6]
    cx, cy, cz = t[6:7], t[7:8], t[8:9]
    abx, aby, abz = t[9:10], t[10:11], t[11:12]
    acx, acy, acz = t[12:13], t[13:14], t[14:15]
    apx, apy, apz = px - ax, py - ay, pz - az
    d1 = abx * apx + aby * apy + abz * apz
    d2 = acx * apx + acy * apy + acz * apz
    bpx, bpy, bpz = px - bx, py - by, pz - bz
    d3 = abx * bpx + aby * bpy + abz * bpz
    d4 = acx * bpx + acy * bpy + acz * bpz
    cpx, cpy, cpz = px - cx, py - cy, pz - cz
    d5 = abx * cpx + aby * cpy + abz * cpz
    d6 = acx * cpx + acy * cpy + acz * cpz
    vc = d1 * d4 - d3 * d2
    vb = d5 * d2 - d1 * d6
    va = d3 * d6 - d5 * d4
    denom = va + vb + vc
    dsafe = jnp.where(jnp.abs(denom) < 1e-12,
                      jnp.where(denom < 0, -1e-12, 1e-12), denom)
    v_in = vb / dsafe
    w_in = vc / dsafe
    u_in = 1.0 - v_in - w_in
    v_ab = _sdiv(d1, d1 - d3)
    w_ac = _sdiv(d2, d2 - d6)
    e1 = d4 - d3
    e2 = d5 - d6
    w_bc = _sdiv(e1, e1 + e2)
    zeros = jnp.zeros_like(d1)
    ones = jnp.ones_like(d1)

    def pick(cond, new, old):
        return tuple(jnp.where(cond, n, o) for n, o in zip(new, old))

    bc = (u_in, v_in, w_in)
    bc = pick((va <= 0) & (e1 >= 0) & (e2 >= 0), (zeros, 1.0 - w_bc, w_bc), bc)
    bc = pick((vb <= 0) & (d2 >= 0) & (d6 <= 0), (1.0 - w_ac, zeros, w_ac), bc)
    bc = pick((vc <= 0) & (d1 >= 0) & (d3 <= 0), (1.0 - v_ab, v_ab, zeros), bc)
    bc = pick((d6 >= 0) & (d5 <= d6), (zeros, zeros, ones), bc)
    bc = pick((d3 >= 0) & (d4 <= d3), (zeros, ones, zeros), bc)
    bc = pick((d1 <= 0) & (d2 <= 0), (ones, zeros, zeros), bc)
    u, v, w = bc
    qx = ax * u + bx * v + cx * w
    qy = ay * u + by * v + cy * w
    qz = az * u + bz * v + cz * w
    dx = qx - px
    dy = qy - py
    dz = qz - pz
    d2sq = dx * dx + dy * dy + dz * dz
    return d2sq, u, v, w


def _search_body(tris_ref, pts_ref, idx_ref, u_ref, v_ref, w_ref,
                 bd_s, bf_s, bu_s, bv_s, bw_s, *, nfaces, nq):
    b = pl.program_id(0)
    fb = pl.program_id(1)
    qi = pl.program_id(2)
    nf = pl.num_programs(1)
    qsl = pl.ds(qi * QB, QB)

    pts = pts_ref[0]
    px = pts[:, 0:1]
    py = pts[:, 1:2]
    pz = pts[:, 2:3]

    first = fb == 0
    bd = jnp.where(first, jnp.float32(1e30), bd_s[qsl])
    bf = jnp.where(first, 0, bf_s[qsl])
    bu = jnp.where(first, 0.0, bu_s[qsl])
    bv = jnp.where(first, 0.0, bv_s[qsl])
    bw = jnp.where(first, 0.0, bw_s[qsl])
    for c in range(FB // LN):
        t = tris_ref[0, :, c * LN:(c + 1) * LN]
        d2sq, u, v, w = _tri_chunk(t, px, py, pz)
        upd = d2sq < bd
        bd = jnp.where(upd, d2sq, bd)
        bf = jnp.where(upd, fb * (FB // LN) + c, bf)
        bu = jnp.where(upd, u, bu)
        bv = jnp.where(upd, v, bv)
        bw = jnp.where(upd, w, bw)
    bd_s[qsl] = bd
    bf_s[qsl] = bf
    bu_s[qsl] = bu
    bv_s[qsl] = bv
    bw_s[qsl] = bw

    @pl.when(fb == nf - 1)
    def _():
        lane = lax.broadcasted_iota(jnp.int32, (QB, LN), 1)
        ff = bf * LN + lane
        m = jnp.min(bd, axis=1, keepdims=True)
        fbig = jnp.where(bd == m, ff, jnp.int32(2147483647))
        fsel = jnp.min(fbig, axis=1, keepdims=True)
        sel = ff == fsel
        ub = jnp.sum(jnp.where(sel, bu, 0.0), axis=1, keepdims=True)
        vb_ = jnp.sum(jnp.where(sel, bv, 0.0), axis=1, keepdims=True)
        wb_ = jnp.sum(jnp.where(sel, bw, 0.0), axis=1, keepdims=True)
        idx_ref[0] = fsel + b * nfaces
        u_ref[0] = jnp.clip(ub, 0.0, 1.0)
        v_ref[0] = jnp.clip(vb_, 0.0, 1.0)
        w_ref[0] = jnp.clip(wb_, 0.0, 1.0)


def _search(tris_t, points):
    bsz, _, nfaces = tris_t.shape
    nq = points.shape[1]
    grid = (bsz, nfaces // FB, nq // QB)
    return pl.pallas_call(
        functools.partial(_search_body, nfaces=nfaces, nq=nq),
        grid=grid,
        in_specs=[
            pl.BlockSpec((1, 15, FB), lambda b, f, q: (b, 0, f)),
            pl.BlockSpec((1, QB, 3), lambda b, f, q: (b, q, 0)),
        ],
        out_specs=[
            pl.BlockSpec((1, QB, 1), lambda b, f, q: (b, q, 0)),
            pl.BlockSpec((1, QB, 1), lambda b, f, q: (b, q, 0)),
            pl.BlockSpec((1, QB, 1), lambda b, f, q: (b, q, 0)),
            pl.BlockSpec((1, QB, 1), lambda b, f, q: (b, q, 0)),
        ],
        out_shape=[
            jax.ShapeDtypeStruct((bsz, nq, 1), jnp.int32),
            jax.ShapeDtypeStruct((bsz, nq, 1), jnp.float32),
            jax.ShapeDtypeStruct((bsz, nq, 1), jnp.float32),
            jax.ShapeDtypeStruct((bsz, nq, 1), jnp.float32),
        ],
        scratch_shapes=[
            pltpu.VMEM((nq, LN), jnp.float32),
            pltpu.VMEM((nq, LN), jnp.int32),
            pltpu.VMEM((nq, LN), jnp.float32),
            pltpu.VMEM((nq, LN), jnp.float32),
            pltpu.VMEM((nq, LN), jnp.float32),
        ],
    )(tris_t, points)


NW = 32
GL = 16


NCOL = 30


def _combine_body(table, idxf, uf, vf, wf, pxf, pyf, pzf,
                  rx_o, ry_o, rz_o, nx_o, ny_o, nz_o, cx_o, cy_o, cz_o, fid_o,
                  idx_v, u_v, v_v, w_v, px_v, py_v, pz_v, idxbuf, cols_v,
                  orx, ory, orz, onx, ony, onz, ocx, ocy, ocz, ofid, sem,
                  *, chunk, nbf):
    cid = lax.axis_index("c")
    sid = lax.axis_index("s")
    wid = sid * 2 + cid
    base = wid * chunk
    pltpu.sync_copy(idxf.at[pl.ds(base, chunk)], idx_v)
    pltpu.sync_copy(uf.at[pl.ds(base, chunk)], u_v)
    pltpu.sync_copy(vf.at[pl.ds(base, chunk)], v_v)
    pltpu.sync_copy(wf.at[pl.ds(base, chunk)], w_v)
    pltpu.sync_copy(pxf.at[pl.ds(base, chunk)], px_v)
    pltpu.sync_copy(pyf.at[pl.ds(base, chunk)], py_v)
    pltpu.sync_copy(pzf.at[pl.ds(base, chunk)], pz_v)
    ngr = chunk // GL
    for g in range(ngr):
        sl = pl.ds(g * GL, GL)
        iv = idx_v[sl]
        for c in range(NCOL):
            idxbuf[c, sl] = iv + c * nbf
    copies = [pltpu.async_copy(table.at[idxbuf.at[c]], cols_v.at[c], sem)
              for c in range(NCOL)]
    for cp in copies:
        cp.wait()
    for g in range(ngr):
        sl = pl.ds(g * GL, GL)
        u16 = u_v[sl]
        v16 = v_v[sl]
        w16 = w_v[sl]

        def gcol(col):
            return cols_v[col, sl]

        p16 = (px_v[sl], py_v[sl], pz_v[sl])
        routs = (orx, ory, orz)
        for k in range(3):
            av = gcol(k)
            bv = gcol(3 + k)
            cv = gcol(6 + k)
            routs[k][sl] = (av * u16 + bv * v16 + cv * w16) - p16[k]
        nouts = (onx, ony, onz)
        for k in range(3):
            av = gcol(9 + k)
            bv = gcol(12 + k)
            cv = gcol(15 + k)
            nouts[k][sl] = av * u16 + bv * v16 + cv * w16
        couts = (ocx, ocy, ocz)
        for k in range(3):
            av = gcol(18 + k)
            bv = gcol(21 + k)
            cv = gcol(24 + k)
            couts[k][sl] = av * u16 + bv * v16 + cv * w16
        f0 = gcol(27)
        f1 = gcol(28)
        f2 = gcol(29)
        m01 = jnp.maximum(u16, v16)
        fidf = jnp.where(w16 > m01, f2, jnp.where(v16 > u16, f1, f0))
        ofid[sl] = fidf.astype(jnp.int32)
    pltpu.sync_copy(orx, rx_o.at[pl.ds(base, chunk)])
    pltpu.sync_copy(ory, ry_o.at[pl.ds(base, chunk)])
    pltpu.sync_copy(orz, rz_o.at[pl.ds(base, chunk)])
    pltpu.sync_copy(onx, nx_o.at[pl.ds(base, chunk)])
    pltpu.sync_copy(ony, ny_o.at[pl.ds(base, chunk)])
    pltpu.sync_copy(onz, nz_o.at[pl.ds(base, chunk)])
    pltpu.sync_copy(ocx, cx_o.at[pl.ds(base, chunk)])
    pltpu.sync_copy(ocy, cy_o.at[pl.ds(base, chunk)])
    pltpu.sync_copy(ocz, cz_o.at[pl.ds(base, chunk)])
    pltpu.sync_copy(ofid, fid_o.at[pl.ds(base, chunk)])


def _combine(table, idxf, uf, vf, wf, pxf, pyf, pzf, nbf):
    bq = idxf.shape[0]
    chunk = bq // NW
    f32 = jnp.float32
    out_type = [jax.ShapeDtypeStruct((bq,), f32)] * 9 + [
        jax.ShapeDtypeStruct((bq,), jnp.int32)]
    scratch = (
        [pltpu.VMEM((chunk,), jnp.int32)]
        + [pltpu.VMEM((chunk,), f32)] * 6
        + [pltpu.VMEM((NCOL, chunk), jnp.int32)]
        + [pltpu.VMEM((NCOL, chunk), f32)]
        + [pltpu.VMEM((chunk,), f32)] * 9
        + [pltpu.VMEM((chunk,), jnp.int32)]
        + [pltpu.SemaphoreType.DMA]
    )
    run = pl.kernel(
        functools.partial(_combine_body, chunk=chunk, nbf=nbf),
        out_type=out_type,
        mesh=plsc.VectorSubcoreMesh(core_axis_name="c", subcore_axis_name="s"),
        scratch_types=scratch,
    )
    return run(table, idxf, uf, vf, wf, pxf, pyf, pzf)


def kernel(triangles, points, normals, cmaps, faces):
    bsz, nfaces = triangles.shape[:2]
    nq = points.shape[1]
    bq = bsz * nq
    t9 = triangles.reshape(bsz, nfaces, 9)
    tris_t = jnp.concatenate(
        [t9, t9[:, :, 3:6] - t9[:, :, 0:3], t9[:, :, 6:9] - t9[:, :, 0:3]],
        axis=2).transpose(0, 2, 1)
    idx_o, u_o, v_o, w_o = _search(tris_t, points)
    nbf = bsz * nfaces
    table = jnp.concatenate([
        triangles.reshape(nbf, 9),
        normals.reshape(nbf, 9),
        cmaps.reshape(nbf, 9),
        faces.reshape(nbf, 3).astype(jnp.float32),
    ], axis=1).T.reshape(NCOL * nbf)
    idxf = idx_o.reshape(bq)
    uf = u_o.reshape(bq)
    vf = v_o.reshape(bq)
    wf = w_o.reshape(bq)
    pxf = points[..., 0].reshape(bq)
    pyf = points[..., 1].reshape(bq)
    pzf = points[..., 2].reshape(bq)
    outs = _combine(table, idxf, uf, vf, wf, pxf, pyf, pzf, nbf)
    rx, ry, rz, nx, ny, nz, cxo, cyo, czo, fid = outs
    residual = jnp.stack([rx, ry, rz], axis=-1).reshape(bsz, nq, 3)
    closest_normals = jnp.stack([nx, ny, nz], axis=-1).reshape(bsz, nq, 3)
    closest_cmaps = jnp.stack([cxo, cyo, czo], axis=-1).reshape(bsz, nq, 3)
    closest_idx = fid.reshape(bsz, nq)
    return residual, closest_normals, closest_cmaps, closest_idx


# speedup vs baseline: 1.2052x; 1.0208x over previous
import functools

import jax
import jax.numpy as jnp
from jax import lax
from jax.experimental import pallas as pl
from jax.experimental.pallas import tpu as pltpu
from jax.experimental.pallas import tpu_sc as plsc

QB = 64
FB = 4096
LN = 128


def _sdiv(n, d):
    ds = jnp.where(jnp.abs(d) < 1e-12, jnp.where(d < 0, -1e-12, 1e-12), d)
    return n / ds


def _tri_chunk(t, px, py, pz):
    ax, ay, az = t[0:1], t[1:2], t[2:3]
    bx, by, bz = t[3:4], t[4:5], t[5:6]
    cx, cy, cz = t[6:7], t[7:8], t[8:9]
    abx, aby, abz = t[9:10], t[10:11], t[11:12]
    acx, acy, acz = t[12:13], t[13:14], t[14:15]
    apx, apy, apz = px - ax, py - ay, pz - az
    d1 = abx * apx + aby * apy + abz * apz
    d2 = acx * apx + acy * apy + acz * apz
    bpx, bpy, bpz = px - bx, py - by, pz - bz
    d3 = abx * bpx + aby * bpy + abz * bpz
    d4 = acx * bpx + acy * bpy + acz * bpz
    cpx, cpy, cpz = px - cx, py - cy, pz - cz
    d5 = abx * cpx + aby * cpy + abz * cpz
    d6 = acx * cpx + acy * cpy + acz * cpz
    vc = d1 * d4 - d3 * d2
    vb = d5 * d2 - d1 * d6
    va = d3 * d6 - d5 * d4
    denom = va + vb + vc
    dsafe = jnp.where(jnp.abs(denom) < 1e-12,
                      jnp.where(denom < 0, -1e-12, 1e-12), denom)
    v_in = vb / dsafe
    w_in = vc / dsafe
    u_in = 1.0 - v_in - w_in
    v_ab = _sdiv(d1, d1 - d3)
    w_ac = _sdiv(d2, d2 - d6)
    e1 = d4 - d3
    e2 = d5 - d6
    w_bc = _sdiv(e1, e1 + e2)
    zeros = jnp.zeros_like(d1)
    ones = jnp.ones_like(d1)

    def pick(cond, new, old):
        return tuple(jnp.where(cond, n, o) for n, o in zip(new, old))

    bc = (u_in, v_in, w_in)
    bc = pick((va <= 0) & (e1 >= 0) & (e2 >= 0), (zeros, 1.0 - w_bc, w_bc), bc)
    bc = pick((vb <= 0) & (d2 >= 0) & (d6 <= 0), (1.0 - w_ac, zeros, w_ac), bc)
    bc = pick((vc <= 0) & (d1 >= 0) & (d3 <= 0), (1.0 - v_ab, v_ab, zeros), bc)
    bc = pick((d6 >= 0) & (d5 <= d6), (zeros, zeros, ones), bc)
    bc = pick((d3 >= 0) & (d4 <= d3), (zeros, ones, zeros), bc)
    bc = pick((d1 <= 0) & (d2 <= 0), (ones, zeros, zeros), bc)
    u, v, w = bc
    qx = ax * u + bx * v + cx * w
    qy = ay * u + by * v + cy * w
    qz = az * u + bz * v + cz * w
    dx = qx - px
    dy = qy - py
    dz = qz - pz
    d2sq = dx * dx + dy * dy + dz * dz
    return d2sq, u, v, w


def _search_body(tris_ref, pts_ref, idx_ref, u_ref, v_ref, w_ref,
                 bd_s, bf_s, bu_s, bv_s, bw_s, *, nfaces, nq):
    b = pl.program_id(0)
    fb = pl.program_id(1)
    qi = pl.program_id(2)
    nf = pl.num_programs(1)
    qsl = pl.ds(qi * QB, QB)

    pts = pts_ref[0]
    px = pts[:, 0:1]
    py = pts[:, 1:2]
    pz = pts[:, 2:3]

    first = fb == 0
    bd = jnp.where(first, jnp.float32(1e30), bd_s[qsl])
    bf = jnp.where(first, 0, bf_s[qsl])
    bu = jnp.where(first, 0.0, bu_s[qsl])
    bv = jnp.where(first, 0.0, bv_s[qsl])
    bw = jnp.where(first, 0.0, bw_s[qsl])
    for c in range(FB // LN):
        t = tris_ref[0, :, c * LN:(c + 1) * LN]
        d2sq, u, v, w = _tri_chunk(t, px, py, pz)
        upd = d2sq < bd
        bd = jnp.where(upd, d2sq, bd)
        bf = jnp.where(upd, fb * (FB // LN) + c, bf)
        bu = jnp.where(upd, u, bu)
        bv = jnp.where(upd, v, bv)
        bw = jnp.where(upd, w, bw)
    bd_s[qsl] = bd
    bf_s[qsl] = bf
    bu_s[qsl] = bu
    bv_s[qsl] = bv
    bw_s[qsl] = bw

    @pl.when(fb == nf - 1)
    def _():
        lane = lax.broadcasted_iota(jnp.int32, (QB, LN), 1)
        ff = bf * LN + lane
        m = jnp.min(bd, axis=1, keepdims=True)
        fbig = jnp.where(bd == m, ff, jnp.int32(2147483647))
        fsel = jnp.min(fbig, axis=1, keepdims=True)
        sel = ff == fsel
        ub = jnp.sum(jnp.where(sel, bu, 0.0), axis=1, keepdims=True)
        vb_ = jnp.sum(jnp.where(sel, bv, 0.0), axis=1, keepdims=True)
        wb_ = jnp.sum(jnp.where(sel, bw, 0.0), axis=1, keepdims=True)
        idx_ref[0] = fsel + b * nfaces
        u_ref[0] = jnp.clip(ub, 0.0, 1.0)
        v_ref[0] = jnp.clip(vb_, 0.0, 1.0)
        w_ref[0] = jnp.clip(wb_, 0.0, 1.0)


def _search(tris_t, points):
    bsz, _, nfaces = tris_t.shape
    nq = points.shape[1]
    grid = (bsz, nfaces // FB, nq // QB)
    return pl.pallas_call(
        functools.partial(_search_body, nfaces=nfaces, nq=nq),
        grid=grid,
        in_specs=[
            pl.BlockSpec((1, 15, FB), lambda b, f, q: (b, 0, f)),
            pl.BlockSpec((1, QB, 3), lambda b, f, q: (b, q, 0)),
        ],
        out_specs=[
            pl.BlockSpec((1, QB, 1), lambda b, f, q: (b, q, 0)),
            pl.BlockSpec((1, QB, 1), lambda b, f, q: (b, q, 0)),
            pl.BlockSpec((1, QB, 1), lambda b, f, q: (b, q, 0)),
            pl.BlockSpec((1, QB, 1), lambda b, f, q: (b, q, 0)),
        ],
        out_shape=[
            jax.ShapeDtypeStruct((bsz, nq, 1), jnp.int32),
            jax.ShapeDtypeStruct((bsz, nq, 1), jnp.float32),
            jax.ShapeDtypeStruct((bsz, nq, 1), jnp.float32),
            jax.ShapeDtypeStruct((bsz, nq, 1), jnp.float32),
        ],
        scratch_shapes=[
            pltpu.VMEM((nq, LN), jnp.float32),
            pltpu.VMEM((nq, LN), jnp.int32),
            pltpu.VMEM((nq, LN), jnp.float32),
            pltpu.VMEM((nq, LN), jnp.float32),
            pltpu.VMEM((nq, LN), jnp.float32),
        ],
    )(tris_t, points)


NW = 32
GL = 16


NCOL = 30


def _combine_body(table, idxf, uf, vf, wf, pxf, pyf, pzf,
                  rx_o, ry_o, rz_o, nx_o, ny_o, nz_o, cx_o, cy_o, cz_o, fid_o,
                  idx_v, u_v, v_v, w_v, px_v, py_v, pz_v, idxbuf, cols_v,
                  orx, ory, orz, onx, ony, onz, ocx, ocy, ocz, ofid, sem,
                  *, chunk, nbf):
    cid = lax.axis_index("c")
    sid = lax.axis_index("s")
    wid = sid * 2 + cid
    base = wid * chunk
    pltpu.sync_copy(idxf.at[pl.ds(base, chunk)], idx_v)
    pltpu.sync_copy(uf.at[pl.ds(base, chunk)], u_v)
    pltpu.sync_copy(vf.at[pl.ds(base, chunk)], v_v)
    pltpu.sync_copy(wf.at[pl.ds(base, chunk)], w_v)
    pltpu.sync_copy(pxf.at[pl.ds(base, chunk)], px_v)
    pltpu.sync_copy(pyf.at[pl.ds(base, chunk)], py_v)
    pltpu.sync_copy(pzf.at[pl.ds(base, chunk)], pz_v)
    ngr = chunk // GL
    for g in range(ngr):
        sl = pl.ds(g * GL, GL)
        iv = idx_v[sl]
        for c in range(NCOL):
            idxbuf[c, sl] = iv + c * nbf
    copies = [pltpu.async_copy(table.at[idxbuf.at[c]], cols_v.at[c], sem)
              for c in range(NCOL)]
    for cp in copies:
        cp.wait()
    for g in range(ngr):
        sl = pl.ds(g * GL, GL)
        u16 = u_v[sl]
        v16 = v_v[sl]
        w16 = w_v[sl]

        def gcol(col):
            return cols_v[col, sl]

        p16 = (px_v[sl], py_v[sl], pz_v[sl])
        routs = (orx, ory, orz)
        for k in range(3):
            av = gcol(k)
            bv = gcol(3 + k)
            cv = gcol(6 + k)
            routs[k][sl] = (av * u16 + bv * v16 + cv * w16) - p16[k]
        nouts = (onx, ony, onz)
        for k in range(3):
            av = gcol(9 + k)
            bv = gcol(12 + k)
            cv = gcol(15 + k)
            nouts[k][sl] = av * u16 + bv * v16 + cv * w16
        couts = (ocx, ocy, ocz)
        for k in range(3):
            av = gcol(18 + k)
            bv = gcol(21 + k)
            cv = gcol(24 + k)
            couts[k][sl] = av * u16 + bv * v16 + cv * w16
        f0 = gcol(27)
        f1 = gcol(28)
        f2 = gcol(29)
        m01 = jnp.maximum(u16, v16)
        fidf = jnp.where(w16 > m01, f2, jnp.where(v16 > u16, f1, f0))
        ofid[sl] = fidf.astype(jnp.int32)
    pltpu.sync_copy(orx, rx_o.at[pl.ds(base, chunk)])
    pltpu.sync_copy(ory, ry_o.at[pl.ds(base, chunk)])
    pltpu.sync_copy(orz, rz_o.at[pl.ds(base, chunk)])
    pltpu.sync_copy(onx, nx_o.at[pl.ds(base, chunk)])
    pltpu.sync_copy(ony, ny_o.at[pl.ds(base, chunk)])
    pltpu.sync_copy(onz, nz_o.at[pl.ds(base, chunk)])
    pltpu.sync_copy(ocx, cx_o.at[pl.ds(base, chunk)])
    pltpu.sync_copy(ocy, cy_o.at[pl.ds(base, chunk)])
    pltpu.sync_copy(ocz, cz_o.at[pl.ds(base, chunk)])
    pltpu.sync_copy(ofid, fid_o.at[pl.ds(base, chunk)])


def _combine(table, idxf, uf, vf, wf, pxf, pyf, pzf, nbf):
    bq = idxf.shape[0]
    chunk = bq // NW
    f32 = jnp.float32
    out_type = [jax.ShapeDtypeStruct((bq,), f32)] * 9 + [
        jax.ShapeDtypeStruct((bq,), jnp.int32)]
    scratch = (
        [pltpu.VMEM((chunk,), jnp.int32)]
        + [pltpu.VMEM((chunk,), f32)] * 6
        + [pltpu.VMEM((NCOL, chunk), jnp.int32)]
        + [pltpu.VMEM((NCOL, chunk), f32)]
        + [pltpu.VMEM((chunk,), f32)] * 9
        + [pltpu.VMEM((chunk,), jnp.int32)]
        + [pltpu.SemaphoreType.DMA]
    )
    run = pl.kernel(
        functools.partial(_combine_body, chunk=chunk, nbf=nbf),
        out_type=out_type,
        mesh=plsc.VectorSubcoreMesh(core_axis_name="c", subcore_axis_name="s"),
        scratch_types=scratch,
    )
    return run(table, idxf, uf, vf, wf, pxf, pyf, pzf)


def kernel(triangles, points, normals, cmaps, faces):
    bsz, nfaces = triangles.shape[:2]
    nq = points.shape[1]
    bq = bsz * nq
    t9 = triangles.reshape(bsz, nfaces, 9)
    tris_t = jnp.concatenate(
        [t9, t9[:, :, 3:6] - t9[:, :, 0:3], t9[:, :, 6:9] - t9[:, :, 0:3]],
        axis=2).transpose(0, 2, 1)
    idx_o, u_o, v_o, w_o = _search(tris_t, points)
    nbf = bsz * nfaces
    table = jnp.concatenate([
        triangles.reshape(nbf, 9),
        normals.reshape(nbf, 9),
        cmaps.reshape(nbf, 9),
        faces.reshape(nbf, 3).astype(jnp.float32),
    ], axis=1).T.reshape(NCOL * nbf)
    idxf = idx_o.reshape(bq)
    uf = u_o.reshape(bq)
    vf = v_o.reshape(bq)
    wf = w_o.reshape(bq)
    pxf = points[..., 0].reshape(bq)
    pyf = points[..., 1].reshape(bq)
    pzf = points[..., 2].reshape(bq)
    outs = _combine(table, idxf, uf, vf, wf, pxf, pyf, pzf, nbf)
    rx, ry, rz, nx, ny, nz, cxo, cyo, czo, fid = outs
    residual = jnp.stack([rx, ry, rz], axis=-1).reshape(bsz, nq, 3)
    closest_normals = jnp.stack([nx, ny, nz], axis=-1).reshape(bsz, nq, 3)
    closest_cmaps = jnp.stack([cxo, cyo, czo], axis=-1).reshape(bsz, nq, 3)
    closest_idx = fid.reshape(bsz, nq)
    return residual, closest_normals, closest_cmaps, closest_idx


# speedup vs baseline: 1.3207x; 1.0959x over previous
import functools

import jax
import jax.numpy as jnp
from jax import lax
from jax.experimental import pallas as pl
from jax.experimental.pallas import tpu as pltpu
from jax.experimental.pallas import tpu_sc as plsc

QB = 64
FB = 8192
LN = 128


def _sdiv(n, d):
    ds = jnp.where(jnp.abs(d) < 1e-12, jnp.where(d < 0, -1e-12, 1e-12), d)
    return n / ds


def _tri_chunk(t, px, py, pz):
    ax, ay, az = t[0:1], t[1:2], t[2:3]
    bx, by, bz = t[3:4], t[4:5], t[5:6]
    cx, cy, cz = t[6:7], t[7:8], t[8:9]
    abx, aby, abz = t[9:10], t[10:11], t[11:12]
    acx, acy, acz = t[12:13], t[13:14], t[14:15]
    apx, apy, apz = px - ax, py - ay, pz - az
    d1 = abx * apx + aby * apy + abz * apz
    d2 = acx * apx + acy * apy + acz * apz
    bpx, bpy, bpz = px - bx, py - by, pz - bz
    d3 = abx * bpx + aby * bpy + abz * bpz
    d4 = acx * bpx + acy * bpy + acz * bpz
    cpx, cpy, cpz = px - cx, py - cy, pz - cz
    d5 = abx * cpx + aby * cpy + abz * cpz
    d6 = acx * cpx + acy * cpy + acz * cpz
    vc = d1 * d4 - d3 * d2
    vb = d5 * d2 - d1 * d6
    va = d3 * d6 - d5 * d4
    denom = va + vb + vc
    dsafe = jnp.where(jnp.abs(denom) < 1e-12,
                      jnp.where(denom < 0, -1e-12, 1e-12), denom)
    v_in = vb / dsafe
    w_in = vc / dsafe
    u_in = 1.0 - v_in - w_in
    v_ab = _sdiv(d1, d1 - d3)
    w_ac = _sdiv(d2, d2 - d6)
    e1 = d4 - d3
    e2 = d5 - d6
    w_bc = _sdiv(e1, e1 + e2)
    zeros = jnp.zeros_like(d1)
    ones = jnp.ones_like(d1)

    def pick(cond, new, old):
        return tuple(jnp.where(cond, n, o) for n, o in zip(new, old))

    bc = (u_in, v_in, w_in)
    bc = pick((va <= 0) & (e1 >= 0) & (e2 >= 0), (zeros, 1.0 - w_bc, w_bc), bc)
    bc = pick((vb <= 0) & (d2 >= 0) & (d6 <= 0), (1.0 - w_ac, zeros, w_ac), bc)
    bc = pick((vc <= 0) & (d1 >= 0) & (d3 <= 0), (1.0 - v_ab, v_ab, zeros), bc)
    bc = pick((d6 >= 0) & (d5 <= d6), (zeros, zeros, ones), bc)
    bc = pick((d3 >= 0) & (d4 <= d3), (zeros, ones, zeros), bc)
    bc = pick((d1 <= 0) & (d2 <= 0), (ones, zeros, zeros), bc)
    u, v, w = bc
    qx = ax * u + bx * v + cx * w
    qy = ay * u + by * v + cy * w
    qz = az * u + bz * v + cz * w
    dx = qx - px
    dy = qy - py
    dz = qz - pz
    d2sq = dx * dx + dy * dy + dz * dz
    return d2sq, u, v, w


def _search_body(tris_ref, pts_ref, idx_ref, u_ref, v_ref, w_ref,
                 bd_s, bf_s, bu_s, bv_s, bw_s, *, nfaces, nq):
    b = pl.program_id(0)
    fb = pl.program_id(1)
    qi = pl.program_id(2)
    nf = pl.num_programs(1)
    qsl = pl.ds(qi * QB, QB)

    pts = pts_ref[0]
    px = pts[:, 0:1]
    py = pts[:, 1:2]
    pz = pts[:, 2:3]

    first = fb == 0
    bd = jnp.where(first, jnp.float32(1e30), bd_s[qsl])
    bf = jnp.where(first, 0, bf_s[qsl])
    bu = jnp.where(first, 0.0, bu_s[qsl])
    bv = jnp.where(first, 0.0, bv_s[qsl])
    bw = jnp.where(first, 0.0, bw_s[qsl])
    for c in range(FB // LN):
        t = tris_ref[0, :, c * LN:(c + 1) * LN]
        d2sq, u, v, w = _tri_chunk(t, px, py, pz)
        upd = d2sq < bd
        bd = jnp.where(upd, d2sq, bd)
        bf = jnp.where(upd, fb * (FB // LN) + c, bf)
        bu = jnp.where(upd, u, bu)
        bv = jnp.where(upd, v, bv)
        bw = jnp.where(upd, w, bw)
    bd_s[qsl] = bd
    bf_s[qsl] = bf
    bu_s[qsl] = bu
    bv_s[qsl] = bv
    bw_s[qsl] = bw

    @pl.when(fb == nf - 1)
    def _():
        lane = lax.broadcasted_iota(jnp.int32, (QB, LN), 1)
        ff = bf * LN + lane
        m = jnp.min(bd, axis=1, keepdims=True)
        fbig = jnp.where(bd == m, ff, jnp.int32(2147483647))
        fsel = jnp.min(fbig, axis=1, keepdims=True)
        sel = ff == fsel
        ub = jnp.sum(jnp.where(sel, bu, 0.0), axis=1, keepdims=True)
        vb_ = jnp.sum(jnp.where(sel, bv, 0.0), axis=1, keepdims=True)
        wb_ = jnp.sum(jnp.where(sel, bw, 0.0), axis=1, keepdims=True)
        idx_ref[0] = fsel + b * nfaces
        u_ref[0] = jnp.clip(ub, 0.0, 1.0)
        v_ref[0] = jnp.clip(vb_, 0.0, 1.0)
        w_ref[0] = jnp.clip(wb_, 0.0, 1.0)


def _search(tris_t, points):
    bsz, _, nfaces = tris_t.shape
    nq = points.shape[1]
    grid = (bsz, nfaces // FB, nq // QB)
    return pl.pallas_call(
        functools.partial(_search_body, nfaces=nfaces, nq=nq),
        grid=grid,
        in_specs=[
            pl.BlockSpec((1, 15, FB), lambda b, f, q: (b, 0, f)),
            pl.BlockSpec((1, QB, 3), lambda b, f, q: (b, q, 0)),
        ],
        out_specs=[
            pl.BlockSpec((1, QB, 1), lambda b, f, q: (b, q, 0)),
            pl.BlockSpec((1, QB, 1), lambda b, f, q: (b, q, 0)),
            pl.BlockSpec((1, QB, 1), lambda b, f, q: (b, q, 0)),
            pl.BlockSpec((1, QB, 1), lambda b, f, q: (b, q, 0)),
        ],
        out_shape=[
            jax.ShapeDtypeStruct((bsz, nq, 1), jnp.int32),
            jax.ShapeDtypeStruct((bsz, nq, 1), jnp.float32),
            jax.ShapeDtypeStruct((bsz, nq, 1), jnp.float32),
            jax.ShapeDtypeStruct((bsz, nq, 1), jnp.float32),
        ],
        scratch_shapes=[
            pltpu.VMEM((nq, LN), jnp.float32),
            pltpu.VMEM((nq, LN), jnp.int32),
            pltpu.VMEM((nq, LN), jnp.float32),
            pltpu.VMEM((nq, LN), jnp.float32),
            pltpu.VMEM((nq, LN), jnp.float32),
        ],
    )(tris_t, points)


NW = 32
GL = 16


NCOL = 30


def _combine_body(table, idxf, uf, vf, wf, pxf, pyf, pzf,
                  rx_o, ry_o, rz_o, nx_o, ny_o, nz_o, cx_o, cy_o, cz_o, fid_o,
                  idx_v, u_v, v_v, w_v, px_v, py_v, pz_v, idxbuf, cols_v,
                  orx, ory, orz, onx, ony, onz, ocx, ocy, ocz, ofid, sem,
                  *, chunk, nbf):
    cid = lax.axis_index("c")
    sid = lax.axis_index("s")
    wid = sid * 2 + cid
    base = wid * chunk
    pltpu.sync_copy(idxf.at[pl.ds(base, chunk)], idx_v)
    pltpu.sync_copy(uf.at[pl.ds(base, chunk)], u_v)
    pltpu.sync_copy(vf.at[pl.ds(base, chunk)], v_v)
    pltpu.sync_copy(wf.at[pl.ds(base, chunk)], w_v)
    pltpu.sync_copy(pxf.at[pl.ds(base, chunk)], px_v)
    pltpu.sync_copy(pyf.at[pl.ds(base, chunk)], py_v)
    pltpu.sync_copy(pzf.at[pl.ds(base, chunk)], pz_v)
    ngr = chunk // GL
    for g in range(ngr):
        sl = pl.ds(g * GL, GL)
        iv = idx_v[sl]
        for c in range(NCOL):
            idxbuf[c, sl] = iv + c * nbf
    copies = [pltpu.async_copy(table.at[idxbuf.at[c]], cols_v.at[c], sem)
              for c in range(NCOL)]
    for cp in copies:
        cp.wait()
    for g in range(ngr):
        sl = pl.ds(g * GL, GL)
        u16 = u_v[sl]
        v16 = v_v[sl]
        w16 = w_v[sl]

        def gcol(col):
            return cols_v[col, sl]

        p16 = (px_v[sl], py_v[sl], pz_v[sl])
        routs = (orx, ory, orz)
        for k in range(3):
            av = gcol(k)
            bv = gcol(3 + k)
            cv = gcol(6 + k)
            routs[k][sl] = (av * u16 + bv * v16 + cv * w16) - p16[k]
        nouts = (onx, ony, onz)
        for k in range(3):
            av = gcol(9 + k)
            bv = gcol(12 + k)
            cv = gcol(15 + k)
            nouts[k][sl] = av * u16 + bv * v16 + cv * w16
        couts = (ocx, ocy, ocz)
        for k in range(3):
            av = gcol(18 + k)
            bv = gcol(21 + k)
            cv = gcol(24 + k)
            couts[k][sl] = av * u16 + bv * v16 + cv * w16
        f0 = gcol(27)
        f1 = gcol(28)
        f2 = gcol(29)
        m01 = jnp.maximum(u16, v16)
        fidf = jnp.where(w16 > m01, f2, jnp.where(v16 > u16, f1, f0))
        ofid[sl] = fidf.astype(jnp.int32)
    pltpu.sync_copy(orx, rx_o.at[pl.ds(base, chunk)])
    pltpu.sync_copy(ory, ry_o.at[pl.ds(base, chunk)])
    pltpu.sync_copy(orz, rz_o.at[pl.ds(base, chunk)])
    pltpu.sync_copy(onx, nx_o.at[pl.ds(base, chunk)])
    pltpu.sync_copy(ony, ny_o.at[pl.ds(base, chunk)])
    pltpu.sync_copy(onz, nz_o.at[pl.ds(base, chunk)])
    pltpu.sync_copy(ocx, cx_o.at[pl.ds(base, chunk)])
    pltpu.sync_copy(ocy, cy_o.at[pl.ds(base, chunk)])
    pltpu.sync_copy(ocz, cz_o.at[pl.ds(base, chunk)])
    pltpu.sync_copy(ofid, fid_o.at[pl.ds(base, chunk)])


def _combine(table, idxf, uf, vf, wf, pxf, pyf, pzf, nbf):
    bq = idxf.shape[0]
    chunk = bq // NW
    f32 = jnp.float32
    out_type = [jax.ShapeDtypeStruct((bq,), f32)] * 9 + [
        jax.ShapeDtypeStruct((bq,), jnp.int32)]
    scratch = (
        [pltpu.VMEM((chunk,), jnp.int32)]
        + [pltpu.VMEM((chunk,), f32)] * 6
        + [pltpu.VMEM((NCOL, chunk), jnp.int32)]
        + [pltpu.VMEM((NCOL, chunk), f32)]
        + [pltpu.VMEM((chunk,), f32)] * 9
        + [pltpu.VMEM((chunk,), jnp.int32)]
        + [pltpu.SemaphoreType.DMA]
    )
    run = pl.kernel(
        functools.partial(_combine_body, chunk=chunk, nbf=nbf),
        out_type=out_type,
        mesh=plsc.VectorSubcoreMesh(core_axis_name="c", subcore_axis_name="s"),
        scratch_types=scratch,
    )
    return run(table, idxf, uf, vf, wf, pxf, pyf, pzf)


def kernel(triangles, points, normals, cmaps, faces):
    bsz, nfaces = triangles.shape[:2]
    nq = points.shape[1]
    bq = bsz * nq
    t9 = triangles.reshape(bsz, nfaces, 9)
    tris_t = jnp.concatenate(
        [t9, t9[:, :, 3:6] - t9[:, :, 0:3], t9[:, :, 6:9] - t9[:, :, 0:3]],
        axis=2).transpose(0, 2, 1)
    idx_o, u_o, v_o, w_o = _search(tris_t, points)
    nbf = bsz * nfaces
    table = jnp.concatenate([
        triangles.reshape(nbf, 9),
        normals.reshape(nbf, 9),
        cmaps.reshape(nbf, 9),
        faces.reshape(nbf, 3).astype(jnp.float32),
    ], axis=1).T.reshape(NCOL * nbf)
    idxf = idx_o.reshape(bq)
    uf = u_o.reshape(bq)
    vf = v_o.reshape(bq)
    wf = w_o.reshape(bq)
    pxf = points[..., 0].reshape(bq)
    pyf = points[..., 1].reshape(bq)
    pzf = points[..., 2].reshape(bq)
    outs = _combine(table, idxf, uf, vf, wf, pxf, pyf, pzf, nbf)
    rx, ry, rz, nx, ny, nz, cxo, cyo, czo, fid = outs
    residual = jnp.stack([rx, ry, rz], axis=-1).reshape(bsz, nq, 3)
    closest_normals = jnp.stack([nx, ny, nz], axis=-1).reshape(bsz, nq, 3)
    closest_cmaps = jnp.stack([cxo, cyo, czo], axis=-1).reshape(bsz, nq, 3)
    closest_idx = fid.reshape(bsz, nq)
    return residual, closest_normals, closest_cmaps, closest_idx
